# baseline (speedup 1.0000x reference)
import functools

import jax
import jax.numpy as jnp
from jax import lax
from jax.experimental import pallas as pl
from jax.experimental.pallas import tpu as pltpu
from jax.experimental.pallas import tpu_sc as plsc

_NC = 2
_NS = 16
_NW = _NC * _NS
_BB = 128
_LANES = 16


@functools.lru_cache(maxsize=None)
def _build(batch, hist, d_model):
  assert batch == _NW * _BB and d_model % 8 == 0 and hist % 2 == 0
  d_tiles = d_model // 8
  mesh = plsc.VectorSubcoreMesh(core_axis_name="c", subcore_axis_name="s")

  @functools.partial(
      pl.kernel,
      out_type=jax.ShapeDtypeStruct((hist, d_tiles, _NW, 8, 128),
                                    jnp.float32),
      mesh=mesh,
      scratch_types=[
          pltpu.VMEM((_BB // 2 * hist,), jnp.int32),
          pltpu.VMEM((hist, _BB), jnp.int32),
          pltpu.VMEM((2, _BB, d_model), jnp.float32),
          pltpu.VMEM((2, d_tiles, 8, 128), jnp.float32),
          pltpu.SemaphoreType.DMA,
          pltpu.SemaphoreType.DMA,
          pltpu.SemaphoreType.DMA,
          pltpu.SemaphoreType.DMA,
      ],
      compiler_params=pltpu.CompilerParams(
          use_tc_tiling_on_sc=False, needs_layout_passes=False),
  )
  def gather_kernel(idx_hbm, table_hbm, out_hbm, idx_raw, ids_v,
                    rows_v, outw_v, gsem0, gsem1, osem0, osem1):
    gsems = (gsem0, gsem1)
    osems = (osem0, osem1)
    wid = lax.axis_index("s") * _NC + lax.axis_index("c")
    iota = lax.iota(jnp.int32, _LANES)

    for half in range(2):
      b0 = half * (_BB // 2)
      pltpu.sync_copy(
          idx_hbm.at[pl.ds((wid * _BB + b0) * hist, _BB // 2 * hist)],
          idx_raw)

      @pl.loop(0, hist)
      def _(h):
        for bc in range(_BB // 2 // _LANES):
          flat = (bc * _LANES + iota) * hist + h
          v = plsc.load_gather(idx_raw, [flat])
          ids_v[h, pl.ds(b0 + bc * _LANES, _LANES)] = v

    def start_gather(h, buf):
      pltpu.async_copy(table_hbm.at[ids_v.at[h]], rows_v.at[buf],
                       gsems[buf])

    def wait_gather(h, buf):
      pltpu.make_async_copy(table_hbm.at[ids_v.at[h]], rows_v.at[buf],
                            gsems[buf]).wait()

    def start_out(h, buf):
      for t in range(d_tiles):
        pltpu.async_copy(outw_v.at[buf, t], out_hbm.at[h, t, wid],
                         osems[buf])

    def wait_out(h, buf):
      for t in range(d_tiles):
        pltpu.make_async_copy(outw_v.at[buf, t], out_hbm.at[h, t, wid],
                              osems[buf]).wait()

    def build(h, buf):
      nbc = _BB // _LANES
      rows16 = [bc * _LANES + iota for bc in range(nbc)]
      zero = jnp.zeros((_LANES,), jnp.int32)
      for t in range(d_tiles):
        for d in range(8):
          cols = zero + (t * 8 + d)
          for bc in range(nbc):
            outw_v[buf, t, d, pl.ds(bc * _LANES, _LANES)] = plsc.load_gather(
                rows_v.at[buf], [rows16[bc], cols])

    start_gather(0, 0)
    start_gather(1, 1)

    @pl.loop(0, hist // 2 - 1)
    def _(hh):
      for b in range(2):
        h = hh * 2 + b

        @pl.when(hh >= 1)
        def _():
          wait_out(h - 2, b)

        wait_gather(h, b)
        build(h, b)
        start_out(h, b)
        start_gather(h + 2, b)

    for b in range(2):
      h = hist - 2 + b
      wait_out(h - 2, b)
      wait_gather(h, b)
      build(h, b)
      start_out(h, b)
    for b in range(2):
      wait_out(hist - 2 + b, b)

  return gather_kernel


def kernel(token_ids, W):
  batch, hist = token_ids.shape
  d_model = W.shape[1]
  ids = token_ids.reshape(-1).astype(jnp.int32)
  x = _build(batch, hist, d_model)(ids, W)
  return x.transpose(2, 4, 0, 1, 3).reshape(batch, hist, d_model)


# speedup vs baseline: 1.5967x; 1.5967x over previous
import functools

import jax
import jax.numpy as jnp
from jax import lax
from jax.experimental import pallas as pl
from jax.experimental.pallas import tpu as pltpu
from jax.experimental.pallas import tpu_sc as plsc

_NC = 2
_NS = 16
_NW = _NC * _NS
_BB = 128
_LANES = 16


@functools.lru_cache(maxsize=None)
def _build(batch, hist, d_model):
  assert batch == _NW * _BB and d_model % 8 == 0 and hist % 2 == 0
  d_tiles = d_model // 8
  mesh = plsc.VectorSubcoreMesh(core_axis_name="c", subcore_axis_name="s")

  @functools.partial(
      pl.kernel,
      out_type=jax.ShapeDtypeStruct((hist, d_tiles, _NW, 8, 128),
                                    jnp.float32),
      mesh=mesh,
      scratch_types=[
          pltpu.VMEM((_BB // 2 * hist,), jnp.int32),
          pltpu.VMEM((hist, _BB), jnp.int32),
          pltpu.VMEM((2, _BB, d_model), jnp.float32),
          pltpu.VMEM((2, d_tiles, 8, 128), jnp.float32),
          pltpu.SemaphoreType.DMA,
          pltpu.SemaphoreType.DMA,
          pltpu.SemaphoreType.DMA,
          pltpu.SemaphoreType.DMA,
      ],
      compiler_params=pltpu.CompilerParams(
          use_tc_tiling_on_sc=False, needs_layout_passes=False),
  )
  def gather_kernel(idx_hbm, table_hbm, out_hbm, idx_raw, ids_v,
                    rows_v, outw_v, gsem0, gsem1, osem0, osem1):
    gsems = (gsem0, gsem1)
    osems = (osem0, osem1)
    wid = lax.axis_index("s") * _NC + lax.axis_index("c")
    iota = lax.iota(jnp.int32, _LANES)

    for half in range(2):
      b0 = half * (_BB // 2)
      pltpu.sync_copy(
          idx_hbm.at[pl.ds((wid * _BB + b0) * hist, _BB // 2 * hist)],
          idx_raw)

      @pl.loop(0, hist)
      def _(h):
        for bc in range(_BB // 2 // _LANES):
          flat = (bc * _LANES + iota) * hist + h
          v = plsc.load_gather(idx_raw, [flat])
          ids_v[h, pl.ds(b0 + bc * _LANES, _LANES)] = v

    def start_gather(h, buf):
      pltpu.async_copy(table_hbm.at[ids_v.at[h]], rows_v.at[buf],
                       gsems[buf])

    def wait_gather(h, buf):
      pltpu.make_async_copy(table_hbm.at[ids_v.at[h]], rows_v.at[buf],
                            gsems[buf]).wait()

    def start_out(h, buf):
      for t in range(d_tiles):
        pltpu.async_copy(outw_v.at[buf, t], out_hbm.at[h, t, wid],
                         osems[buf])

    def wait_out(h, buf):
      for t in range(d_tiles):
        pltpu.make_async_copy(outw_v.at[buf, t], out_hbm.at[h, t, wid],
                              osems[buf]).wait()

    def build(h, buf):
      nbc = _BB // _LANES
      rows16 = [bc * _LANES + iota for bc in range(nbc)]
      zero = jnp.zeros((_LANES,), jnp.int32)

      @plsc.parallel_loop(0, d_model, unroll=4)
      def _(dd):
        t = lax.shift_right_logical(dd, 3)
        d = lax.bitwise_and(dd, 7)
        cols = zero + dd
        for bc in range(nbc):
          vals = plsc.load_gather(rows_v.at[buf], [rows16[bc], cols])
          outw_v[buf, t, d, pl.ds(bc * _LANES, _LANES)] = vals

    start_gather(0, 0)
    start_gather(1, 1)

    @pl.loop(0, hist // 2 - 1)
    def _(hh):
      for b in range(2):
        h = hh * 2 + b

        @pl.when(hh >= 1)
        def _():
          wait_out(h - 2, b)

        wait_gather(h, b)
        build(h, b)
        start_out(h, b)
        start_gather(h + 2, b)

    for b in range(2):
      h = hist - 2 + b
      wait_out(h - 2, b)
      wait_gather(h, b)
      build(h, b)
      start_out(h, b)
    for b in range(2):
      wait_out(hist - 2 + b, b)

  return gather_kernel


def kernel(token_ids, W):
  batch, hist = token_ids.shape
  d_model = W.shape[1]
  ids = token_ids.reshape(-1).astype(jnp.int32)
  x = _build(batch, hist, d_model)(ids, W)
  return x.transpose(2, 4, 0, 1, 3).reshape(batch, hist, d_model)


# speedup vs baseline: 1.6049x; 1.0052x over previous
import functools

import jax
import jax.numpy as jnp
from jax import lax
from jax.experimental import pallas as pl
from jax.experimental.pallas import tpu as pltpu
from jax.experimental.pallas import tpu_sc as plsc

_NC = 2
_NS = 16
_NW = _NC * _NS
_BB = 128
_LANES = 16


@functools.lru_cache(maxsize=None)
def _build(batch, hist, d_model):
  assert batch == _NW * _BB and d_model % 8 == 0 and hist % 2 == 0
  d_tiles = d_model // 8
  mesh = plsc.VectorSubcoreMesh(core_axis_name="c", subcore_axis_name="s")

  @functools.partial(
      pl.kernel,
      out_type=jax.ShapeDtypeStruct((hist, d_tiles, _NW, 8, 128),
                                    jnp.float32),
      mesh=mesh,
      scratch_types=[
          pltpu.VMEM((_BB // 2 * hist,), jnp.int32),
          pltpu.VMEM((hist, _BB), jnp.int32),
          pltpu.VMEM((2, _BB, d_model), jnp.float32),
          pltpu.VMEM((2, d_tiles, 8, 128), jnp.float32),
          pltpu.SemaphoreType.DMA,
          pltpu.SemaphoreType.DMA,
          pltpu.SemaphoreType.DMA,
          pltpu.SemaphoreType.DMA,
      ],
      compiler_params=pltpu.CompilerParams(
          use_tc_tiling_on_sc=False, needs_layout_passes=False),
  )
  def gather_kernel(idx_hbm, table_hbm, out_hbm, idx_raw, ids_v,
                    rows_v, outw_v, gsem0, gsem1, osem0, osem1):
    gsems = (gsem0, gsem1)
    osems = (osem0, osem1)
    wid = lax.axis_index("s") * _NC + lax.axis_index("c")
    iota = lax.iota(jnp.int32, _LANES)

    for half in range(2):
      b0 = half * (_BB // 2)
      pltpu.sync_copy(
          idx_hbm.at[pl.ds((wid * _BB + b0) * hist, _BB // 2 * hist)],
          idx_raw)

      @pl.loop(0, hist)
      def _(h):
        for bc in range(_BB // 2 // _LANES):
          flat = (bc * _LANES + iota) * hist + h
          v = plsc.load_gather(idx_raw, [flat])
          ids_v[h, pl.ds(b0 + bc * _LANES, _LANES)] = v

    def start_gather(h, buf):
      pltpu.async_copy(table_hbm.at[ids_v.at[h]], rows_v.at[buf],
                       gsems[buf])

    def wait_gather(h, buf):
      pltpu.make_async_copy(table_hbm.at[ids_v.at[h]], rows_v.at[buf],
                            gsems[buf]).wait()

    def start_out(h, buf):
      pltpu.async_copy(outw_v.at[buf], out_hbm.at[h, :, wid], osems[buf])

    def wait_out(h, buf):
      pltpu.make_async_copy(outw_v.at[buf], out_hbm.at[h, :, wid],
                            osems[buf]).wait()

    def build(h, buf):
      nbc = _BB // _LANES
      rows16 = [bc * _LANES + iota for bc in range(nbc)]
      zero = jnp.zeros((_LANES,), jnp.int32)

      @plsc.parallel_loop(0, d_model, unroll=4)
      def _(dd):
        t = lax.shift_right_logical(dd, 3)
        d = lax.bitwise_and(dd, 7)
        cols = zero + dd
        for bc in range(nbc):
          vals = plsc.load_gather(rows_v.at[buf], [rows16[bc], cols])
          outw_v[buf, t, d, pl.ds(bc * _LANES, _LANES)] = vals

    start_gather(0, 0)
    start_gather(1, 1)

    @pl.loop(0, hist // 2 - 1)
    def _(hh):
      for b in range(2):
        h = hh * 2 + b

        @pl.when(hh >= 1)
        def _():
          wait_out(h - 2, b)

        wait_gather(h, b)
        build(h, b)
        start_out(h, b)
        start_gather(h + 2, b)

    for b in range(2):
      h = hist - 2 + b
      wait_out(h - 2, b)
      wait_gather(h, b)
      build(h, b)
      start_out(h, b)
    for b in range(2):
      wait_out(hist - 2 + b, b)

  return gather_kernel


def kernel(token_ids, W):
  batch, hist = token_ids.shape
  d_model = W.shape[1]
  ids = token_ids.reshape(-1).astype(jnp.int32)
  x = _build(batch, hist, d_model)(ids, W)
  return x.transpose(2, 4, 0, 1, 3).reshape(batch, hist, d_model)
